# baseline (speedup 1.0000x reference)
import math

import jax
import jax.numpy as jnp
from jax.experimental import pallas as pl
from jax.experimental.pallas import tpu as pltpu

_C0 = math.sqrt(2.0 / math.pi)
_C1 = 0.044715


def _gelu_block(x_ref, o_ref):
    x = x_ref[...]
    u = _C0 * (x + _C1 * x * x * x)
    o_ref[...] = 0.5 * x * (1.0 + jnp.tanh(u))


def kernel(x, log_k_inject):
    B, T, D = x.shape
    xf = x.reshape(B * T, D)
    blk = 512
    grid = (B * T // blk,)
    y = pl.pallas_call(
        _gelu_block,
        out_shape=jax.ShapeDtypeStruct((B * T, D), x.dtype),
        grid=grid,
        in_specs=[pl.BlockSpec((blk, D), lambda i: (i, 0))],
        out_specs=pl.BlockSpec((blk, D), lambda i: (i, 0)),
        compiler_params=pltpu.CompilerParams(
            dimension_semantics=("arbitrary",),
        ),
    )(xf)
    return y.reshape(B, T, D)


# speedup vs baseline: 1.0310x; 1.0310x over previous
import math

import jax
import jax.numpy as jnp
from jax.experimental import pallas as pl
from jax.experimental.pallas import tpu as pltpu

_C0 = math.sqrt(2.0 / math.pi)
_C1 = 0.044715


def _gelu_block(x_ref, o_ref):
    x = x_ref[...]
    hx = 0.5 * x
    u = x * (_C0 + (_C0 * _C1) * (x * x))
    t = jnp.tanh(u)
    o_ref[...] = hx + hx * t


def kernel(x, log_k_inject):
    B, T, D = x.shape
    xf = x.reshape(B * T, D)
    blk = 1024
    grid = (B * T // blk,)
    y = pl.pallas_call(
        _gelu_block,
        out_shape=jax.ShapeDtypeStruct((B * T, D), x.dtype),
        grid=grid,
        in_specs=[pl.BlockSpec((blk, D), lambda i: (i, 0))],
        out_specs=pl.BlockSpec((blk, D), lambda i: (i, 0)),
        compiler_params=pltpu.CompilerParams(
            dimension_semantics=("arbitrary",),
        ),
    )(xf)
    return y.reshape(B, T, D)
